# baseline (speedup 1.0000x reference)
import functools

import jax
import jax.numpy as jnp
from jax import lax
from jax.experimental import pallas as pl
from jax.experimental.pallas import tpu as pltpu
from jax.experimental.pallas import tpu_sc as plsc

_BINS = 10
_NBIN = 26214
_NBPAD = 26624
_SLICE = _NBPAD // 16
_ROWS = 128


def _tc_body(x_ref, t_ref, nll_ref, bin_ref, *, nbin):
    xb = x_ref[0]
    t = t_ref[0]
    m = jnp.max(xb, axis=0)
    s = jnp.sum(jnp.exp(xb - m[None]), axis=0)
    lse = jnp.log(s) + m
    cids = lax.broadcasted_iota(jnp.int32, xb.shape, 0)
    xt = jnp.sum(jnp.where(cids == t[None], xb, 0.0), axis=0)
    logp_t = xt - lse
    g = jnp.abs(jnp.exp(logp_t) - 1.0)
    b = jnp.floor(g * (nbin - 0.0001)).astype(jnp.int32)
    b = jnp.minimum(b, nbin - 1)
    bin_ref[0] = b.reshape(8, 128, 128)
    nll_ref[0] = (-logp_t).reshape(8, 128, 128)


def _tc_stage(x, target, nbin, wave):
    B, C, H, W = x.shape
    Hb = 256
    return pl.pallas_call(
        functools.partial(_tc_body, nbin=nbin),
        grid=(2, H // Hb),
        in_specs=[
            pl.BlockSpec((1, C, Hb, W),
                         lambda i, j: (2 * wave + i, 0, j, 0)),
            pl.BlockSpec((1, Hb, W), lambda i, j: (2 * wave + i, j, 0)),
        ],
        out_specs=[
            pl.BlockSpec((1, 8, 128, 128), lambda i, j: (i, j, 0, 0)),
            pl.BlockSpec((1, 8, 128, 128), lambda i, j: (i, j, 0, 0)),
        ],
        out_shape=[
            jax.ShapeDtypeStruct((2, 16, 128, 128), jnp.float32),
            jax.ShapeDtypeStruct((2, 16, 128, 128), jnp.int32),
        ],
        compiler_params=pltpu.CompilerParams(
            dimension_semantics=("parallel", "parallel"),
        ),
    )(x, target)


def _sc_body(bin_hbm, nll_hbm, out_hbm,
             idx_v, nll_v, ones_v, zer_v, redc_v, reds_v, row_v, part_v,
             cnt_sh, sum_sh, part_sh, sem):
    c = lax.axis_index("c")
    s = lax.axis_index("s")

    zeros16 = jnp.zeros((16,), jnp.float32)
    ones16 = jnp.ones((16,), jnp.float32)

    def _fill(i, _):
        zer_v[pl.ds(i * 16, 16)] = zeros16
        return 0
    lax.fori_loop(0, _SLICE // 16, _fill, 0)

    def _fill1(i, _):
        ones_v[pl.ds(i * 16, 16)] = ones16
        return 0
    lax.fori_loop(0, 8, _fill1, 0)

    off = s * _SLICE
    pltpu.sync_copy(zer_v, cnt_sh.at[pl.ds(off, _SLICE)])
    pltpu.sync_copy(zer_v, sum_sh.at[pl.ds(off, _SLICE)])

    pltpu.sync_copy(bin_hbm.at[c, s], idx_v)
    pltpu.sync_copy(nll_hbm.at[c, s], nll_v)
    plsc.subcore_barrier()

    K = 8

    def _scat(rr, _):
        cops = []
        for b in range(K):
            r = rr * K + b
            irow = idx_v.at[r]
            cops.append(pltpu.async_copy(ones_v, cnt_sh.at[irow],
                                         sem, add=True))
            cops.append(pltpu.async_copy(nll_v.at[r], sum_sh.at[irow],
                                         sem, add=True))
        for cop in cops:
            cop.wait()
        return 0

    lax.fori_loop(0, _ROWS // K, _scat, 0)
    plsc.subcore_barrier()

    pltpu.sync_copy(cnt_sh.at[pl.ds(off, _SLICE)], redc_v)
    pltpu.sync_copy(sum_sh.at[pl.ds(off, _SLICE)], reds_v)

    def _red(i, carry):
        acc_s, acc_n = carry
        cv = redc_v[pl.ds(i * 16, 16)]
        sv = reds_v[pl.ds(i * 16, 16)]
        acc_s = acc_s + sv / jnp.maximum(cv, 1.0)
        acc_n = acc_n + jnp.where(cv > 0.0, 1.0, 0.0)
        return acc_s, acc_n

    acc_s, acc_n = lax.fori_loop(0, _SLICE // 16, _red, (zeros16, zeros16))
    row_v[0] = acc_s
    row_v[1] = acc_n
    pltpu.sync_copy(row_v, part_sh.at[s])
    plsc.subcore_barrier()

    @pl.when(s == 0)
    def _():
        pltpu.sync_copy(part_sh, part_v)

        def _fold(jj, carry):
            fa, fb = carry
            return fa + part_v[jj, 0], fb + part_v[jj, 1]

        acc_s, acc_n = lax.fori_loop(0, 16, _fold, (zeros16, zeros16))
        row_v[0] = acc_s
        row_v[1] = acc_n
        pltpu.sync_copy(row_v, out_hbm.at[c])


@functools.lru_cache(maxsize=1)
def _make_sc_hist():
    @functools.partial(
        pl.kernel,
        out_type=jax.ShapeDtypeStruct((2, 2, 16), jnp.float32),
        mesh=plsc.VectorSubcoreMesh(core_axis_name="c", subcore_axis_name="s",
                                    num_cores=2, num_subcores=16),
        scratch_types=[
            pltpu.VMEM((_ROWS, 128), jnp.int32),
            pltpu.VMEM((_ROWS, 128), jnp.float32),
            pltpu.VMEM((128,), jnp.float32),
            pltpu.VMEM((_SLICE,), jnp.float32),
            pltpu.VMEM((_SLICE,), jnp.float32),
            pltpu.VMEM((_SLICE,), jnp.float32),
            pltpu.VMEM((2, 16), jnp.float32),
            pltpu.VMEM((16, 2, 16), jnp.float32),
            pltpu.VMEM_SHARED((_NBPAD,), jnp.float32),
            pltpu.VMEM_SHARED((_NBPAD,), jnp.float32),
            pltpu.VMEM_SHARED((16, 2, 16), jnp.float32),
            pltpu.SemaphoreType.DMA,
        ],
    )
    def _sc_hist(bin_hbm, nll_hbm, out_hbm, *rest):
        _sc_body(bin_hbm, nll_hbm, out_hbm, *rest)

    return _sc_hist


def kernel(x, target):
    B, C, H, W = x.shape
    N = H * W
    nbin = N // _BINS
    assert (B, C, H, W) == (4, 19, 512, 512) and nbin == _NBIN

    sc = _make_sc_hist()
    outs = []
    for w in range(2):
        nll4, bin4 = _tc_stage(x, target, nbin, w)
        outs.append(sc(bin4, nll4))
    out = jnp.concatenate(outs, axis=0)
    s_b = jnp.sum(out[:, 0, :], axis=-1)
    nz_b = jnp.sum(out[:, 1, :], axis=-1)
    return jnp.mean(s_b / nz_b)


# speedup vs baseline: 1.0244x; 1.0244x over previous
import functools

import jax
import jax.numpy as jnp
from jax import lax
from jax.experimental import pallas as pl
from jax.experimental.pallas import tpu as pltpu
from jax.experimental.pallas import tpu_sc as plsc

_BINS = 10
_NBIN = 26214
_NBPAD = 26624
_SLICE = _NBPAD // 16
_ROWS = 128


def _tc_body(x_ref, t_ref, nll_ref, bin_ref, *, nbin):
    xb = x_ref[0]
    t = t_ref[0]
    m = jnp.max(xb, axis=0)
    s = jnp.sum(jnp.exp(xb - m[None]), axis=0)
    lse = jnp.log(s) + m
    cids = lax.broadcasted_iota(jnp.int32, xb.shape, 0)
    xt = jnp.sum(jnp.where(cids == t[None], xb, 0.0), axis=0)
    logp_t = xt - lse
    g = jnp.abs(jnp.exp(logp_t) - 1.0)
    b = jnp.floor(g * (nbin - 0.0001)).astype(jnp.int32)
    b = jnp.minimum(b, nbin - 1)
    bin_ref[0] = b.reshape(8, 128, 128)
    nll_ref[0] = (-logp_t).reshape(8, 128, 128)


def _tc_stage(x, target, nbin, wave):
    B, C, H, W = x.shape
    Hb = 256
    return pl.pallas_call(
        functools.partial(_tc_body, nbin=nbin),
        grid=(2, H // Hb),
        in_specs=[
            pl.BlockSpec((1, C, Hb, W),
                         lambda i, j: (2 * wave + i, 0, j, 0)),
            pl.BlockSpec((1, Hb, W), lambda i, j: (2 * wave + i, j, 0)),
        ],
        out_specs=[
            pl.BlockSpec((1, 8, 128, 128), lambda i, j: (i, j, 0, 0)),
            pl.BlockSpec((1, 8, 128, 128), lambda i, j: (i, j, 0, 0)),
        ],
        out_shape=[
            jax.ShapeDtypeStruct((2, 16, 128, 128), jnp.float32),
            jax.ShapeDtypeStruct((2, 16, 128, 128), jnp.int32),
        ],
        compiler_params=pltpu.CompilerParams(
            dimension_semantics=("parallel", "parallel"),
        ),
    )(x, target)


def _sc_body(bin_hbm, nll_hbm, out_hbm,
             idx_v, nll_v, ones_v, zer_v, redc_v, reds_v, row_v, part_v,
             cnt_sh, sum_sh, part_sh, sem):
    c = lax.axis_index("c")
    s = lax.axis_index("s")

    zeros16 = jnp.zeros((16,), jnp.float32)
    ones16 = jnp.ones((16,), jnp.float32)

    def _fill(i, _):
        zer_v[pl.ds(i * 16, 16)] = zeros16
        return 0
    lax.fori_loop(0, _SLICE // 16, _fill, 0)

    def _fill1(i, _):
        ones_v[pl.ds(i * 16, 16)] = ones16
        return 0
    lax.fori_loop(0, 8, _fill1, 0)

    off = s * _SLICE
    pltpu.sync_copy(zer_v, cnt_sh.at[pl.ds(off, _SLICE)])
    pltpu.sync_copy(zer_v, sum_sh.at[pl.ds(off, _SLICE)])

    pltpu.sync_copy(bin_hbm.at[c, s], idx_v)
    pltpu.sync_copy(nll_hbm.at[c, s], nll_v)
    plsc.subcore_barrier()

    K = 8

    def _grp(rr, fire):
        for b in range(K):
            r = rr * K + b
            irow = idx_v.at[r]
            if fire:
                pltpu.async_copy(ones_v, cnt_sh.at[irow], sem, add=True)
                pltpu.async_copy(nll_v.at[r], sum_sh.at[irow], sem, add=True)
            else:
                pltpu.make_async_copy(ones_v, cnt_sh.at[irow], sem).wait()
                pltpu.make_async_copy(nll_v.at[r], sum_sh.at[irow],
                                      sem).wait()

    def _scat(rr, _):
        _grp(rr, True)

        @pl.when(rr > 0)
        def _():
            _grp(rr - 1, False)

        return 0

    lax.fori_loop(0, _ROWS // K, _scat, 0)
    _grp(_ROWS // K - 1, False)
    plsc.subcore_barrier()

    pltpu.sync_copy(cnt_sh.at[pl.ds(off, _SLICE)], redc_v)
    pltpu.sync_copy(sum_sh.at[pl.ds(off, _SLICE)], reds_v)

    def _red(i, carry):
        acc_s, acc_n = carry
        cv = redc_v[pl.ds(i * 16, 16)]
        sv = reds_v[pl.ds(i * 16, 16)]
        acc_s = acc_s + sv / jnp.maximum(cv, 1.0)
        acc_n = acc_n + jnp.where(cv > 0.0, 1.0, 0.0)
        return acc_s, acc_n

    acc_s, acc_n = lax.fori_loop(0, _SLICE // 16, _red, (zeros16, zeros16))
    row_v[0] = acc_s
    row_v[1] = acc_n
    pltpu.sync_copy(row_v, part_sh.at[s])
    plsc.subcore_barrier()

    @pl.when(s == 0)
    def _():
        pltpu.sync_copy(part_sh, part_v)

        def _fold(jj, carry):
            fa, fb = carry
            return fa + part_v[jj, 0], fb + part_v[jj, 1]

        acc_s, acc_n = lax.fori_loop(0, 16, _fold, (zeros16, zeros16))
        row_v[0] = acc_s
        row_v[1] = acc_n
        pltpu.sync_copy(row_v, out_hbm.at[c])


@functools.lru_cache(maxsize=1)
def _make_sc_hist():
    @functools.partial(
        pl.kernel,
        out_type=jax.ShapeDtypeStruct((2, 2, 16), jnp.float32),
        mesh=plsc.VectorSubcoreMesh(core_axis_name="c", subcore_axis_name="s",
                                    num_cores=2, num_subcores=16),
        scratch_types=[
            pltpu.VMEM((_ROWS, 128), jnp.int32),
            pltpu.VMEM((_ROWS, 128), jnp.float32),
            pltpu.VMEM((128,), jnp.float32),
            pltpu.VMEM((_SLICE,), jnp.float32),
            pltpu.VMEM((_SLICE,), jnp.float32),
            pltpu.VMEM((_SLICE,), jnp.float32),
            pltpu.VMEM((2, 16), jnp.float32),
            pltpu.VMEM((16, 2, 16), jnp.float32),
            pltpu.VMEM_SHARED((_NBPAD,), jnp.float32),
            pltpu.VMEM_SHARED((_NBPAD,), jnp.float32),
            pltpu.VMEM_SHARED((16, 2, 16), jnp.float32),
            pltpu.SemaphoreType.DMA,
        ],
    )
    def _sc_hist(bin_hbm, nll_hbm, out_hbm, *rest):
        _sc_body(bin_hbm, nll_hbm, out_hbm, *rest)

    return _sc_hist


def kernel(x, target):
    B, C, H, W = x.shape
    N = H * W
    nbin = N // _BINS
    assert (B, C, H, W) == (4, 19, 512, 512) and nbin == _NBIN

    sc = _make_sc_hist()
    outs = []
    for w in range(2):
        nll4, bin4 = _tc_stage(x, target, nbin, w)
        outs.append(sc(bin4, nll4))
    out = jnp.concatenate(outs, axis=0)
    s_b = jnp.sum(out[:, 0, :], axis=-1)
    nz_b = jnp.sum(out[:, 1, :], axis=-1)
    return jnp.mean(s_b / nz_b)


# speedup vs baseline: 1.0245x; 1.0001x over previous
import functools

import jax
import jax.numpy as jnp
from jax import lax
from jax.experimental import pallas as pl
from jax.experimental.pallas import tpu as pltpu
from jax.experimental.pallas import tpu_sc as plsc

_BINS = 10
_NBIN = 26214
_NBPAD = 26624
_SLICE = _NBPAD // 16
_ROWS = 128


def _tc_body(x_ref, t_ref, nll_ref, bin_ref, *, nbin):
    xb = x_ref[0]
    t = t_ref[0]
    m = jnp.max(xb, axis=0)
    s = jnp.sum(jnp.exp(xb - m[None]), axis=0)
    lse = jnp.log(s) + m
    cids = lax.broadcasted_iota(jnp.int32, xb.shape, 0)
    xt = jnp.sum(jnp.where(cids == t[None], xb, 0.0), axis=0)
    logp_t = xt - lse
    g = jnp.abs(jnp.exp(logp_t) - 1.0)
    b = jnp.floor(g * (nbin - 0.0001)).astype(jnp.int32)
    b = jnp.minimum(b, nbin - 1)
    bin_ref[0] = b.reshape(8, 128, 128)
    nll_ref[0] = (-logp_t).reshape(8, 128, 128)


def _tc_stage(x, target, nbin, wave):
    B, C, H, W = x.shape
    Hb = 256
    return pl.pallas_call(
        functools.partial(_tc_body, nbin=nbin),
        grid=(2, H // Hb),
        in_specs=[
            pl.BlockSpec((1, C, Hb, W),
                         lambda i, j: (2 * wave + i, 0, j, 0)),
            pl.BlockSpec((1, Hb, W), lambda i, j: (2 * wave + i, j, 0)),
        ],
        out_specs=[
            pl.BlockSpec((1, 8, 128, 128), lambda i, j: (i, j, 0, 0)),
            pl.BlockSpec((1, 8, 128, 128), lambda i, j: (i, j, 0, 0)),
        ],
        out_shape=[
            jax.ShapeDtypeStruct((2, 16, 128, 128), jnp.float32),
            jax.ShapeDtypeStruct((2, 16, 128, 128), jnp.int32),
        ],
        compiler_params=pltpu.CompilerParams(
            dimension_semantics=("parallel", "parallel"),
        ),
    )(x, target)


def _sc_body(bin_hbm, nll_hbm, out_hbm,
             idx_v, nll_v, ones_v, zer_v, redc_v, reds_v, row_v, part_v,
             cnt_sh, sum_sh, part_sh, sem):
    c = lax.axis_index("c")
    s = lax.axis_index("s")

    zeros16 = jnp.zeros((16,), jnp.float32)
    ones16 = jnp.ones((16,), jnp.float32)

    def _fill(i, _):
        zer_v[pl.ds(i * 16, 16)] = zeros16
        return 0
    lax.fori_loop(0, _SLICE // 16, _fill, 0)

    def _fill1(i, _):
        ones_v[pl.ds(i * 16, 16)] = ones16
        return 0
    lax.fori_loop(0, 8, _fill1, 0)

    off = s * _SLICE
    pltpu.sync_copy(zer_v, cnt_sh.at[pl.ds(off, _SLICE)])
    pltpu.sync_copy(zer_v, sum_sh.at[pl.ds(off, _SLICE)])

    pltpu.sync_copy(bin_hbm.at[c, s], idx_v)
    pltpu.sync_copy(nll_hbm.at[c, s], nll_v)
    plsc.subcore_barrier()

    K = 16

    def _grp(rr, fire):
        for b in range(K):
            r = rr * K + b
            irow = idx_v.at[r]
            if fire:
                pltpu.async_copy(ones_v, cnt_sh.at[irow], sem, add=True)
                pltpu.async_copy(nll_v.at[r], sum_sh.at[irow], sem, add=True)
            else:
                pltpu.make_async_copy(ones_v, cnt_sh.at[irow], sem).wait()
                pltpu.make_async_copy(nll_v.at[r], sum_sh.at[irow],
                                      sem).wait()

    def _scat(rr, _):
        _grp(rr, True)

        @pl.when(rr > 0)
        def _():
            _grp(rr - 1, False)

        return 0

    lax.fori_loop(0, _ROWS // K, _scat, 0)
    _grp(_ROWS // K - 1, False)
    plsc.subcore_barrier()

    pltpu.sync_copy(cnt_sh.at[pl.ds(off, _SLICE)], redc_v)
    pltpu.sync_copy(sum_sh.at[pl.ds(off, _SLICE)], reds_v)

    def _red(i, carry):
        acc_s, acc_n = carry
        cv = redc_v[pl.ds(i * 16, 16)]
        sv = reds_v[pl.ds(i * 16, 16)]
        acc_s = acc_s + sv / jnp.maximum(cv, 1.0)
        acc_n = acc_n + jnp.where(cv > 0.0, 1.0, 0.0)
        return acc_s, acc_n

    acc_s, acc_n = lax.fori_loop(0, _SLICE // 16, _red, (zeros16, zeros16))
    row_v[0] = acc_s
    row_v[1] = acc_n
    pltpu.sync_copy(row_v, part_sh.at[s])
    plsc.subcore_barrier()

    @pl.when(s == 0)
    def _():
        pltpu.sync_copy(part_sh, part_v)

        def _fold(jj, carry):
            fa, fb = carry
            return fa + part_v[jj, 0], fb + part_v[jj, 1]

        acc_s, acc_n = lax.fori_loop(0, 16, _fold, (zeros16, zeros16))
        row_v[0] = acc_s
        row_v[1] = acc_n
        pltpu.sync_copy(row_v, out_hbm.at[c])


@functools.lru_cache(maxsize=1)
def _make_sc_hist():
    @functools.partial(
        pl.kernel,
        out_type=jax.ShapeDtypeStruct((2, 2, 16), jnp.float32),
        mesh=plsc.VectorSubcoreMesh(core_axis_name="c", subcore_axis_name="s",
                                    num_cores=2, num_subcores=16),
        scratch_types=[
            pltpu.VMEM((_ROWS, 128), jnp.int32),
            pltpu.VMEM((_ROWS, 128), jnp.float32),
            pltpu.VMEM((128,), jnp.float32),
            pltpu.VMEM((_SLICE,), jnp.float32),
            pltpu.VMEM((_SLICE,), jnp.float32),
            pltpu.VMEM((_SLICE,), jnp.float32),
            pltpu.VMEM((2, 16), jnp.float32),
            pltpu.VMEM((16, 2, 16), jnp.float32),
            pltpu.VMEM_SHARED((_NBPAD,), jnp.float32),
            pltpu.VMEM_SHARED((_NBPAD,), jnp.float32),
            pltpu.VMEM_SHARED((16, 2, 16), jnp.float32),
            pltpu.SemaphoreType.DMA,
        ],
    )
    def _sc_hist(bin_hbm, nll_hbm, out_hbm, *rest):
        _sc_body(bin_hbm, nll_hbm, out_hbm, *rest)

    return _sc_hist


def kernel(x, target):
    B, C, H, W = x.shape
    N = H * W
    nbin = N // _BINS
    assert (B, C, H, W) == (4, 19, 512, 512) and nbin == _NBIN

    sc = _make_sc_hist()
    outs = []
    for w in range(2):
        nll4, bin4 = _tc_stage(x, target, nbin, w)
        outs.append(sc(bin4, nll4))
    out = jnp.concatenate(outs, axis=0)
    s_b = jnp.sum(out[:, 0, :], axis=-1)
    nz_b = jnp.sum(out[:, 1, :], axis=-1)
    return jnp.mean(s_b / nz_b)
